# baseline (speedup 1.0000x reference)
import functools

import jax
import jax.numpy as jnp
from jax import lax
from jax.experimental import pallas as pl
from jax.experimental.pallas import tpu as pltpu
from jax.experimental.pallas import tpu_sc as plsc

_NC = 2
_NS = 16
_NW = _NC * _NS
_LANES = 16


def _row_segments(L):
    segs = []
    off = 0
    while off < L:
        n = min(128, L - off)
        segs.append((off, n))
        off += n
    return segs


def _transpose_table(tab_t, tail_rm, V, E):
    TK = 512
    PR = TK * E // 128
    NQ = V // TK
    TAIL = V - NQ * TK
    MPW = (NQ + _NW - 1) // _NW
    RPAD = (MPW * _NW * TK + TAIL) * E // 128 + 128

    mesh = plsc.VectorSubcoreMesh(core_axis_name="c", subcore_axis_name="s")

    @functools.partial(
        pl.kernel,
        out_type=jax.ShapeDtypeStruct((RPAD, 128), jnp.float32),
        mesh=mesh,
        compiler_params=pltpu.CompilerParams(
            use_tc_tiling_on_sc=True, needs_layout_passes=False),
        scratch_types=[
            pltpu.VMEM((E, TK), jnp.float32),
            pltpu.VMEM((E, TK), jnp.float32),
            pltpu.VMEM((PR, 128), jnp.float32),
            pltpu.VMEM((PR, 128), jnp.float32),
            pltpu.SemaphoreType.DMA,
            pltpu.SemaphoreType.DMA,
            pltpu.SemaphoreType.DMA,
            pltpu.SemaphoreType.DMA,
        ],
    )
    def transpose(tab_hbm, tail_hbm, out_hbm, in_a, in_b, out_a, out_b,
                  isem_a, isem_b, osem_a, osem_b):
        w = lax.axis_index("s") * _NC + lax.axis_index("c")

        iota = lax.iota(jnp.int32, _LANES)
        rbase = lax.shift_right_logical(iota, 2)
        cbase = lax.shift_left(jnp.bitwise_and(iota, 3), 5)

        def in_copy(q, buf, sem):
            return pltpu.make_async_copy(
                tab_hbm.at[:, pl.ds(q * TK, TK)], buf, sem)

        def out_copy(q, buf, sem):
            return pltpu.make_async_copy(
                buf, out_hbm.at[pl.ds(q * PR, PR)], sem)

        def tbody(src, dst):
            @pl.loop(0, TK // _LANES)
            def _g(g):
                ridx = rbase + (g * (_LANES * E // 128))
                for f in range(E):
                    vals = src[f, pl.ds(g * _LANES, _LANES)]
                    plsc.store_scatter(dst, [ridx, cbase + f], vals)

        def chunk_of(m):
            return w + m * _NW

        in_copy(chunk_of(0), in_a, isem_a).start()

        @pl.loop(0, MPW - 2, step=2)
        def _pair(m):
            q0 = chunk_of(m)
            q1 = chunk_of(m + 1)
            in_copy(q1, in_b, isem_b).start()
            in_copy(q0, in_a, isem_a).wait()
            tbody(in_a, out_a)
            out_copy(q0, out_a, osem_a).start()
            in_copy(chunk_of(m + 2), in_a, isem_a).start()
            in_copy(q1, in_b, isem_b).wait()
            tbody(in_b, out_b)
            out_copy(q1, out_b, osem_b).start()
            out_copy(q0, out_a, osem_a).wait()
            out_copy(q1, out_b, osem_b).wait()

        m = MPW - 2
        q0 = chunk_of(m)
        q1 = chunk_of(m + 1)

        @pl.when(q1 * TK <= V - TK)
        def _():
            in_copy(q1, in_b, isem_b).start()
        in_copy(q0, in_a, isem_a).wait()
        tbody(in_a, out_a)
        out_copy(q0, out_a, osem_a).start()

        @pl.when(q1 * TK <= V - TK)
        def _():
            in_copy(q1, in_b, isem_b).wait()
            tbody(in_b, out_b)
            out_copy(q1, out_b, osem_b).start()
            out_copy(q1, out_b, osem_b).wait()
        out_copy(q0, out_a, osem_a).wait()

        if TAIL:
            tpr = TAIL * E // 128

            @pl.when(w == 0)
            def _():
                pltpu.sync_copy(tail_hbm, out_a.at[pl.ds(0, tpr)])
                pltpu.sync_copy(out_a.at[pl.ds(0, tpr)],
                                out_hbm.at[pl.ds(NQ * PR, tpr)])

    return transpose(tab_t, tail_rm), RPAD


def _pool_sums(x, table_rm, B, L, E):
    R = B // _NW
    CB = 4
    NCHUNK = R // CB
    EG = E // _LANES
    U = 4
    segs = _row_segments(L)
    assert NCHUNK % 2 == 0 and L % U == 0

    mesh = plsc.VectorSubcoreMesh(core_axis_name="c", subcore_axis_name="s")

    @functools.partial(
        pl.kernel,
        out_type=jax.ShapeDtypeStruct((B, E), jnp.float32),
        mesh=mesh,
        compiler_params=pltpu.CompilerParams(use_tc_tiling_on_sc=False),
        scratch_types=[
            pltpu.VMEM((R, L), jnp.int32),
            pltpu.VMEM((CB * L, E), jnp.float32),
            pltpu.VMEM((CB * L, E), jnp.float32),
            pltpu.VMEM((R, E), jnp.float32),
            pltpu.SemaphoreType.DMA,
            pltpu.SemaphoreType.DMA,
        ],
    )
    def pool(x_hbm, tab_hbm, out_hbm, idx_v, rows_a, rows_b, acc_v, sem_a, sem_b):
        w = lax.axis_index("s") * _NC + lax.axis_index("c")
        row0 = w * R

        def copies(c, buf, sem):
            out = []
            for b in range(CB):
                for (o, n) in segs:
                    src = tab_hbm.at[idx_v.at[c * CB + b, pl.ds(o, n)]]
                    dst = buf.at[pl.ds(b * L + o, n)]
                    out.append(pltpu.make_async_copy(src, dst, sem))
            return out

        def fire(c, buf, sem):
            for d in copies(c, buf, sem):
                d.start()

        def drain(c, buf, sem):
            for d in copies(c, buf, sem):
                d.wait()

        def compute(c, buf):
            for b in range(CB):
                base = b * L

                def body(j, accs, base=base):
                    r = base + j * U
                    out = list(accs)
                    for g in range(EG):
                        s = pl.ds(g * _LANES, _LANES)
                        out[2 * g] = out[2 * g] + buf[r, s] + buf[r + 1, s]
                        out[2 * g + 1] = out[2 * g + 1] + buf[r + 2, s] + buf[r + 3, s]
                    return tuple(out)

                accs = lax.fori_loop(
                    0, L // U, body,
                    tuple(jnp.zeros((_LANES,), jnp.float32) for _ in range(2 * EG)))
                row = c * CB + b
                for g in range(EG):
                    acc_v[row, pl.ds(g * _LANES, _LANES)] = accs[2 * g] + accs[2 * g + 1]

        pltpu.sync_copy(x_hbm.at[pl.ds(row0, R)], idx_v)
        fire(0, rows_a, sem_a)

        @pl.loop(0, NCHUNK - 2, step=2)
        def _pair(c0):
            fire(c0 + 1, rows_b, sem_b)
            drain(c0, rows_a, sem_a)
            compute(c0, rows_a)
            fire(c0 + 2, rows_a, sem_a)
            drain(c0 + 1, rows_b, sem_b)
            compute(c0 + 1, rows_b)

        c0 = NCHUNK - 2
        fire(c0 + 1, rows_b, sem_b)
        drain(c0, rows_a, sem_a)
        compute(c0, rows_a)
        drain(c0 + 1, rows_b, sem_b)
        compute(c0 + 1, rows_b)

        pltpu.sync_copy(acc_v, out_hbm.at[pl.ds(row0, R)])

    return pool(x, table_rm)


def _mlp(pooled, w1t, b1r, w2t, b2r, inv_l):
    B = pooled.shape[0]
    OUT = w2t.shape[1]

    def body(s_ref, w1_ref, b1_ref, w2_ref, b2_ref, o_ref):
        h = s_ref[...] * inv_l
        h = jnp.dot(h, w1_ref[...], preferred_element_type=jnp.float32) + b1_ref[...]
        h = jnp.maximum(h, 0.0)
        o = jnp.dot(h, w2_ref[...], preferred_element_type=jnp.float32) + b2_ref[...]
        o_ref[...] = 1.0 / (1.0 + jnp.exp(-o))

    return pl.pallas_call(
        body,
        out_shape=jax.ShapeDtypeStruct((B, OUT), jnp.float32),
    )(pooled, w1t, b1r, w2t, b2r)


def kernel(x, table, W1, b1, W2, b2):
    B, L = x.shape
    V, E = table.shape
    HID = W1.shape[0]
    OUT = W2.shape[0]
    assert B % _NW == 0 and L % 8 == 0 and E % _LANES == 0 and (128 % E) == 0

    TK = 512
    nq = V // TK
    tail = V - nq * TK
    tail_rm = table[V - tail:, :].reshape(tail * E // 128, 128)
    t2, rpad = _transpose_table(table.T, tail_rm, V, E)
    table_rm = t2.reshape(rpad * (128 // E), E)
    pooled = _pool_sums(x.astype(jnp.int32), table_rm, B, L, E)
    return _mlp(
        pooled,
        W1.T,
        b1.reshape(1, HID),
        W2.T,
        b2.reshape(1, OUT),
        1.0 / L,
    )


# speedup vs baseline: 1.2517x; 1.2517x over previous
import functools

import jax
import jax.numpy as jnp
from jax import lax
from jax.experimental import pallas as pl
from jax.experimental.pallas import tpu as pltpu
from jax.experimental.pallas import tpu_sc as plsc

_NC = 2
_NS = 16
_NW = _NC * _NS
_LANES = 16


def _row_segments(L):
    segs = []
    off = 0
    while off < L:
        n = min(128, L - off)
        segs.append((off, n))
        off += n
    return segs


def _transpose_table(tab_t, tail_rm, V, E):
    TK = 512
    PR = TK * E // 128
    NQ = V // TK
    TAIL = V - NQ * TK
    MPW = (NQ + _NW - 1) // _NW
    RPAD = (MPW * _NW * TK + TAIL) * E // 128 + 128

    mesh = plsc.VectorSubcoreMesh(core_axis_name="c", subcore_axis_name="s")

    @functools.partial(
        pl.kernel,
        out_type=jax.ShapeDtypeStruct((RPAD, 128), jnp.float32),
        mesh=mesh,
        compiler_params=pltpu.CompilerParams(
            use_tc_tiling_on_sc=True, needs_layout_passes=False),
        scratch_types=[
            pltpu.VMEM((E, TK), jnp.float32),
            pltpu.VMEM((E, TK), jnp.float32),
            pltpu.VMEM((PR, 128), jnp.float32),
            pltpu.VMEM((PR, 128), jnp.float32),
            pltpu.SemaphoreType.DMA,
            pltpu.SemaphoreType.DMA,
            pltpu.SemaphoreType.DMA,
            pltpu.SemaphoreType.DMA,
        ],
    )
    def transpose(tab_hbm, tail_hbm, out_hbm, in_a, in_b, out_a, out_b,
                  isem_a, isem_b, osem_a, osem_b):
        w = lax.axis_index("s") * _NC + lax.axis_index("c")

        iota = lax.iota(jnp.int32, _LANES)
        rbase = lax.shift_right_logical(iota, 2)
        cbase = lax.shift_left(jnp.bitwise_and(iota, 3), 5)

        def in_copy(q, buf, sem):
            return pltpu.make_async_copy(
                tab_hbm.at[:, pl.ds(q * TK, TK)], buf, sem)

        def out_copy(q, buf, sem):
            return pltpu.make_async_copy(
                buf, out_hbm.at[pl.ds(q * PR, PR)], sem)

        def tbody(src, dst):
            @plsc.parallel_loop(0, TK // _LANES, unroll=2)
            def _g(g):
                ridx = rbase + (g * (_LANES * E // 128))
                for f in range(E):
                    vals = src[f, pl.ds(g * _LANES, _LANES)]
                    plsc.store_scatter(dst, [ridx, cbase + f], vals)

        def chunk_of(m):
            return w + m * _NW

        in_copy(chunk_of(0), in_a, isem_a).start()

        @pl.loop(0, MPW - 2, step=2)
        def _pair(m):
            q0 = chunk_of(m)
            q1 = chunk_of(m + 1)
            in_copy(q1, in_b, isem_b).start()
            in_copy(q0, in_a, isem_a).wait()
            tbody(in_a, out_a)
            out_copy(q0, out_a, osem_a).start()
            in_copy(chunk_of(m + 2), in_a, isem_a).start()
            in_copy(q1, in_b, isem_b).wait()
            tbody(in_b, out_b)
            out_copy(q1, out_b, osem_b).start()
            out_copy(q0, out_a, osem_a).wait()
            out_copy(q1, out_b, osem_b).wait()

        m = MPW - 2
        q0 = chunk_of(m)
        q1 = chunk_of(m + 1)

        @pl.when(q1 * TK <= V - TK)
        def _():
            in_copy(q1, in_b, isem_b).start()
        in_copy(q0, in_a, isem_a).wait()
        tbody(in_a, out_a)
        out_copy(q0, out_a, osem_a).start()

        @pl.when(q1 * TK <= V - TK)
        def _():
            in_copy(q1, in_b, isem_b).wait()
            tbody(in_b, out_b)
            out_copy(q1, out_b, osem_b).start()
            out_copy(q1, out_b, osem_b).wait()
        out_copy(q0, out_a, osem_a).wait()

        if TAIL:
            tpr = TAIL * E // 128

            @pl.when(w == 0)
            def _():
                pltpu.sync_copy(tail_hbm, out_a.at[pl.ds(0, tpr)])
                pltpu.sync_copy(out_a.at[pl.ds(0, tpr)],
                                out_hbm.at[pl.ds(NQ * PR, tpr)])

    return transpose(tab_t, tail_rm), RPAD


def _pool_sums(x, table_rm, B, L, E):
    R = B // _NW
    CB = 4
    NCHUNK = R // CB
    EG = E // _LANES
    U = 4
    segs = _row_segments(L)
    assert NCHUNK % 2 == 0 and L % U == 0

    mesh = plsc.VectorSubcoreMesh(core_axis_name="c", subcore_axis_name="s")

    @functools.partial(
        pl.kernel,
        out_type=jax.ShapeDtypeStruct((B, E), jnp.float32),
        mesh=mesh,
        compiler_params=pltpu.CompilerParams(use_tc_tiling_on_sc=False),
        scratch_types=[
            pltpu.VMEM((R, L), jnp.int32),
            pltpu.VMEM((CB * L, E), jnp.float32),
            pltpu.VMEM((CB * L, E), jnp.float32),
            pltpu.VMEM((R, E), jnp.float32),
            pltpu.SemaphoreType.DMA,
            pltpu.SemaphoreType.DMA,
        ],
    )
    def pool(x_hbm, tab_hbm, out_hbm, idx_v, rows_a, rows_b, acc_v, sem_a, sem_b):
        w = lax.axis_index("s") * _NC + lax.axis_index("c")
        row0 = w * R

        def copies(c, buf, sem):
            out = []
            for b in range(CB):
                for (o, n) in segs:
                    src = tab_hbm.at[idx_v.at[c * CB + b, pl.ds(o, n)]]
                    dst = buf.at[pl.ds(b * L + o, n)]
                    out.append(pltpu.make_async_copy(src, dst, sem))
            return out

        def fire(c, buf, sem):
            for d in copies(c, buf, sem):
                d.start()

        def drain(c, buf, sem):
            for d in copies(c, buf, sem):
                d.wait()

        def compute(c, buf):
            for b in range(CB):
                base = b * L

                def body(j, accs, base=base):
                    r = base + j * U
                    out = list(accs)
                    for g in range(EG):
                        s = pl.ds(g * _LANES, _LANES)
                        out[2 * g] = out[2 * g] + buf[r, s] + buf[r + 1, s]
                        out[2 * g + 1] = out[2 * g + 1] + buf[r + 2, s] + buf[r + 3, s]
                    return tuple(out)

                accs = lax.fori_loop(
                    0, L // U, body,
                    tuple(jnp.zeros((_LANES,), jnp.float32) for _ in range(2 * EG)))
                row = c * CB + b
                for g in range(EG):
                    acc_v[row, pl.ds(g * _LANES, _LANES)] = accs[2 * g] + accs[2 * g + 1]

        pltpu.sync_copy(x_hbm.at[pl.ds(row0, R)], idx_v)
        fire(0, rows_a, sem_a)

        @pl.loop(0, NCHUNK - 2, step=2)
        def _pair(c0):
            fire(c0 + 1, rows_b, sem_b)
            drain(c0, rows_a, sem_a)
            compute(c0, rows_a)
            fire(c0 + 2, rows_a, sem_a)
            drain(c0 + 1, rows_b, sem_b)
            compute(c0 + 1, rows_b)

        c0 = NCHUNK - 2
        fire(c0 + 1, rows_b, sem_b)
        drain(c0, rows_a, sem_a)
        compute(c0, rows_a)
        drain(c0 + 1, rows_b, sem_b)
        compute(c0 + 1, rows_b)

        pltpu.sync_copy(acc_v, out_hbm.at[pl.ds(row0, R)])

    return pool(x, table_rm)


def _mlp(pooled, w1t, b1r, w2t, b2r, inv_l):
    B = pooled.shape[0]
    OUT = w2t.shape[1]

    def body(s_ref, w1_ref, b1_ref, w2_ref, b2_ref, o_ref):
        h = s_ref[...] * inv_l
        h = jnp.dot(h, w1_ref[...], preferred_element_type=jnp.float32) + b1_ref[...]
        h = jnp.maximum(h, 0.0)
        o = jnp.dot(h, w2_ref[...], preferred_element_type=jnp.float32) + b2_ref[...]
        o_ref[...] = 1.0 / (1.0 + jnp.exp(-o))

    return pl.pallas_call(
        body,
        out_shape=jax.ShapeDtypeStruct((B, OUT), jnp.float32),
    )(pooled, w1t, b1r, w2t, b2r)


def kernel(x, table, W1, b1, W2, b2):
    B, L = x.shape
    V, E = table.shape
    HID = W1.shape[0]
    OUT = W2.shape[0]
    assert B % _NW == 0 and L % 8 == 0 and E % _LANES == 0 and (128 % E) == 0

    TK = 512
    nq = V // TK
    tail = V - nq * TK
    tail_rm = table[V - tail:, :].reshape(tail * E // 128, 128)
    t2, rpad = _transpose_table(table.T, tail_rm, V, E)
    table_rm = t2.reshape(rpad * (128 // E), E)
    pooled = _pool_sums(x.astype(jnp.int32), table_rm, B, L, E)
    return _mlp(
        pooled,
        W1.T,
        b1.reshape(1, HID),
        W2.T,
        b2.reshape(1, OUT),
        1.0 / L,
    )
